# baseline (speedup 1.0000x reference)
import functools

import jax
import jax.numpy as jnp
from jax import lax
from jax.experimental import pallas as pl
from jax.experimental.pallas import tpu as pltpu
from jax.experimental.pallas import tpu_sc as plsc

_NC = 2
_NS = 16
_NW = _NC * _NS
_GROUP = 8
_LANES = 16


def _sc_gather_sum(idx_pad, table, n_valid):
    vocab, d = table.shape
    mesh = plsc.VectorSubcoreMesh(core_axis_name="c", subcore_axis_name="s")

    def body(idx_hbm, table_hbm, out_hbm, idx_v, rows_v, acc_v, sem):
        wid = lax.axis_index("s") * _NC + lax.axis_index("c")
        base = wid * _GROUP
        pltpu.sync_copy(idx_hbm.at[pl.ds(base, _GROUP)], idx_v)
        pltpu.async_copy(table_hbm.at[idx_v], rows_v, sem).wait()
        for c in range(d // _LANES):
            sl = pl.ds(c * _LANES, _LANES)
            acc = jnp.zeros((_LANES,), jnp.float32)
            for r in range(_GROUP):
                w_r = jnp.where(base + r < n_valid, 1.0, 0.0)
                acc = acc + rows_v[r, sl] * w_r
            acc_v[sl] = acc
        pltpu.sync_copy(acc_v, out_hbm.at[wid])

    run = pl.kernel(
        body,
        out_type=jax.ShapeDtypeStruct((_NW, d), jnp.float32),
        mesh=mesh,
        scratch_types=[
            pltpu.VMEM((_GROUP,), jnp.int32),
            pltpu.VMEM((_GROUP, d), jnp.float32),
            pltpu.VMEM((d,), jnp.float32),
            pltpu.SemaphoreType.DMA,
        ],
    )
    return run(idx_pad, table)


def _tc_body(nt, tv, vocab, part_ref, wp_ref, bp_ref, wo_ref, bo_ref,
             out_ref, logit_sc, h_sc, m_ref):
    i = pl.program_id(0)

    @pl.when(i == 0)
    def _init():
        e = jnp.sum(part_ref[...], axis=0, keepdims=True)
        e8 = jnp.broadcast_to(e, (8, e.shape[1]))
        h8 = jnp.dot(e8, wp_ref[...], preferred_element_type=jnp.float32)
        h_sc[...] = jnp.maximum(h8 + bp_ref[...], 0.0)
        m_ref[0] = -jnp.inf

    lt = jnp.dot(h_sc[...], wo_ref[...], preferred_element_type=jnp.float32)
    logits = lt[0:1, :] + bo_ref[...]
    off = pl.multiple_of(i * tv, 128)
    logit_sc[0:1, pl.ds(off, tv)] = logits
    col = i * tv + lax.broadcasted_iota(jnp.int32, (1, tv), 1)
    lm = jnp.where(col < vocab, logits, -jnp.inf)
    m_ref[0] = jnp.maximum(m_ref[0], jnp.max(lm))

    @pl.when(i == nt - 1)
    def _finish():
        m = m_ref[0]

        def body_j(j, acc):
            o = pl.multiple_of(j * tv, 128)
            ch = logit_sc[0:1, pl.ds(o, tv)]
            cc = j * tv + lax.broadcasted_iota(jnp.int32, (1, tv), 1)
            chm = jnp.where(cc < vocab, ch, -jnp.inf)
            return acc + jnp.sum(jnp.exp(chm - m))

        s = lax.fori_loop(0, nt, body_j, 0.0)
        lse = m + jnp.max(jnp.log(jnp.full((1, 128), s)))
        out_ref[...] = logit_sc[0:1, :vocab] - lse


def _tc_mlp_logsoftmax(partials, W_proj, b_proj2, W_out, b_out2):
    d, vocab = W_out.shape
    tv = 4096
    nt = -(-vocab // tv)

    return pl.pallas_call(
        functools.partial(_tc_body, nt, tv, vocab),
        grid=(nt,),
        in_specs=[
            pl.BlockSpec((_NW, d), lambda i: (0, 0)),
            pl.BlockSpec((d, d), lambda i: (0, 0)),
            pl.BlockSpec((1, d), lambda i: (0, 0)),
            pl.BlockSpec((d, tv), lambda i: (0, i)),
            pl.BlockSpec((1, tv), lambda i: (0, i)),
        ],
        out_specs=pl.BlockSpec((1, vocab), lambda i: (0, 0)),
        out_shape=jax.ShapeDtypeStruct((1, vocab), jnp.float32),
        scratch_shapes=[
            pltpu.VMEM((1, nt * tv), jnp.float32),
            pltpu.VMEM((8, d), jnp.float32),
            pltpu.SMEM((1,), jnp.float32),
        ],
        compiler_params=pltpu.CompilerParams(
            dimension_semantics=("arbitrary",),
        ),
    )(partials, W_proj, b_proj2, W_out, b_out2)


def kernel(inputs, table, W_proj, b_proj, W_out, b_out):
    n = inputs.shape[0]
    idx = inputs.astype(jnp.int32)
    n_pad = _NW * _GROUP
    idx_pad = jnp.pad(idx, (0, n_pad - n))
    partials = _sc_gather_sum(idx_pad, table, n)
    return _tc_mlp_logsoftmax(
        partials,
        W_proj,
        b_proj.reshape(1, -1),
        W_out,
        b_out.reshape(1, -1),
    )


# speedup vs baseline: 1.0003x; 1.0003x over previous
import functools

import jax
import jax.numpy as jnp
from jax import lax
from jax.experimental import pallas as pl
from jax.experimental.pallas import tpu as pltpu
from jax.experimental.pallas import tpu_sc as plsc

_NC = 2
_NS = 16
_NW = _NC * _NS
_GROUP = 8
_LANES = 16


def _sc_gather_sum(idx_pad, table, n_valid):
    vocab, d = table.shape
    mesh = plsc.VectorSubcoreMesh(core_axis_name="c", subcore_axis_name="s")

    def body(idx_hbm, table_hbm, out_hbm, idx_v, rows_v, acc_v, sem):
        wid = lax.axis_index("s") * _NC + lax.axis_index("c")
        base = wid * _GROUP
        pltpu.sync_copy(idx_hbm.at[pl.ds(base, _GROUP)], idx_v)
        pltpu.async_copy(table_hbm.at[idx_v], rows_v, sem).wait()
        for c in range(d // _LANES):
            sl = pl.ds(c * _LANES, _LANES)
            acc = jnp.zeros((_LANES,), jnp.float32)
            for r in range(_GROUP):
                w_r = jnp.where(base + r < n_valid, 1.0, 0.0)
                acc = acc + rows_v[r, sl] * w_r
            acc_v[sl] = acc
        pltpu.sync_copy(acc_v, out_hbm.at[wid])

    run = pl.kernel(
        body,
        out_type=jax.ShapeDtypeStruct((_NW, d), jnp.float32),
        mesh=mesh,
        scratch_types=[
            pltpu.VMEM((_GROUP,), jnp.int32),
            pltpu.VMEM((_GROUP, d), jnp.float32),
            pltpu.VMEM((d,), jnp.float32),
            pltpu.SemaphoreType.DMA,
        ],
        compiler_params=pltpu.CompilerParams(use_tc_tiling_on_sc=True),
    )
    return run(idx_pad, table)


def _tc_body(nt, tv, vocab, part_ref, wp_ref, bp_ref, wo_ref, bo_ref,
             out_ref, logit_sc, h_sc, m_ref):
    i = pl.program_id(0)

    @pl.when(i == 0)
    def _init():
        e = jnp.sum(part_ref[...], axis=0, keepdims=True)
        e8 = jnp.broadcast_to(e, (8, e.shape[1]))
        h8 = jnp.dot(e8, wp_ref[...], preferred_element_type=jnp.float32)
        h_sc[...] = jnp.maximum(h8 + bp_ref[...], 0.0)
        m_ref[0] = -jnp.inf

    lt = jnp.dot(h_sc[...], wo_ref[...], preferred_element_type=jnp.float32)
    logits = lt[0:1, :] + bo_ref[...]
    off = pl.multiple_of(i * tv, 128)
    logit_sc[0:1, pl.ds(off, tv)] = logits
    col = i * tv + lax.broadcasted_iota(jnp.int32, (1, tv), 1)
    lm = jnp.where(col < vocab, logits, -jnp.inf)
    m_ref[0] = jnp.maximum(m_ref[0], jnp.max(lm))

    @pl.when(i == nt - 1)
    def _finish():
        m = m_ref[0]

        def body_j(j, acc):
            o = pl.multiple_of(j * tv, 128)
            ch = logit_sc[0:1, pl.ds(o, tv)]
            cc = j * tv + lax.broadcasted_iota(jnp.int32, (1, tv), 1)
            chm = jnp.where(cc < vocab, ch, -jnp.inf)
            return acc + jnp.sum(jnp.exp(chm - m))

        s = lax.fori_loop(0, nt, body_j, 0.0)
        lse = m + jnp.max(jnp.log(jnp.full((1, 128), s)))
        out_ref[...] = logit_sc[0:1, :vocab] - lse


def _tc_mlp_logsoftmax(partials, W_proj, b_proj2, W_out, b_out2):
    d, vocab = W_out.shape
    tv = 4096
    nt = -(-vocab // tv)

    return pl.pallas_call(
        functools.partial(_tc_body, nt, tv, vocab),
        grid=(nt,),
        in_specs=[
            pl.BlockSpec((_NW, d), lambda i: (0, 0)),
            pl.BlockSpec((d, d), lambda i: (0, 0)),
            pl.BlockSpec((1, d), lambda i: (0, 0)),
            pl.BlockSpec((d, tv), lambda i: (0, i)),
            pl.BlockSpec((1, tv), lambda i: (0, i)),
        ],
        out_specs=pl.BlockSpec((1, vocab), lambda i: (0, 0)),
        out_shape=jax.ShapeDtypeStruct((1, vocab), jnp.float32),
        scratch_shapes=[
            pltpu.VMEM((1, nt * tv), jnp.float32),
            pltpu.VMEM((8, d), jnp.float32),
            pltpu.SMEM((1,), jnp.float32),
        ],
        compiler_params=pltpu.CompilerParams(
            dimension_semantics=("arbitrary",),
        ),
    )(partials, W_proj, b_proj2, W_out, b_out2)


def kernel(inputs, table, W_proj, b_proj, W_out, b_out):
    n = inputs.shape[0]
    idx = inputs.astype(jnp.int32)
    n_pad = _NW * _GROUP
    idx_pad = jnp.pad(idx, (0, n_pad - n))
    partials = _sc_gather_sum(idx_pad, table, n)
    return _tc_mlp_logsoftmax(
        partials,
        W_proj,
        b_proj.reshape(1, -1),
        W_out,
        b_out.reshape(1, -1),
    )


# speedup vs baseline: 1.6157x; 1.6152x over previous
import functools

import jax
import jax.numpy as jnp
from jax import lax
from jax.experimental import pallas as pl
from jax.experimental.pallas import tpu as pltpu
from jax.experimental.pallas import tpu_sc as plsc

_NC = 2
_NS = 16
_NW = _NC * _NS
_GROUP = 8
_LANES = 16


def _sc_gather_sum(idx_pad, table, n_valid):
    vocab, d = table.shape
    mesh = plsc.VectorSubcoreMesh(core_axis_name="c", subcore_axis_name="s")

    def body(idx_hbm, table_hbm, out_hbm, idx_v, rows_v, acc_v, sem):
        wid = lax.axis_index("s") * _NC + lax.axis_index("c")
        base = wid * _GROUP
        pltpu.sync_copy(idx_hbm.at[pl.ds(base, _GROUP)], idx_v)
        pltpu.async_copy(table_hbm.at[idx_v], rows_v, sem).wait()
        for c in range(d // _LANES):
            sl = pl.ds(c * _LANES, _LANES)
            acc = jnp.zeros((_LANES,), jnp.float32)
            for r in range(_GROUP):
                w_r = jnp.where(base + r < n_valid, 1.0, 0.0)
                acc = acc + rows_v[r, sl] * w_r
            acc_v[sl] = acc
        pltpu.sync_copy(acc_v, out_hbm.at[wid])

    run = pl.kernel(
        body,
        out_type=jax.ShapeDtypeStruct((_NW, d), jnp.float32),
        mesh=mesh,
        scratch_types=[
            pltpu.VMEM((_GROUP,), jnp.int32),
            pltpu.VMEM((_GROUP, d), jnp.float32),
            pltpu.VMEM((d,), jnp.float32),
            pltpu.SemaphoreType.DMA,
        ],
    )
    return run(idx_pad, table)


def _a_body(nt, tv, vocab, pT_ref, wpT_ref, bpT_ref, wo_ref, bo_ref,
            lg_ref, lse_ref, h_sc, m_ref, s_ref):
    i = pl.program_id(0)

    @pl.when(i == 0)
    def _init():
        e = jnp.sum(pT_ref[...], axis=1, keepdims=True)
        e8 = jnp.broadcast_to(e, (e.shape[0], 8))
        h = jnp.dot(wpT_ref[...], e8, preferred_element_type=jnp.float32)
        h_sc[...] = jnp.maximum(h + bpT_ref[...], 0.0)
        m_ref[0] = -jnp.inf
        s_ref[0] = 0.0

    lt = jnp.dot(wo_ref[...], h_sc[...], preferred_element_type=jnp.float32)
    logits = jnp.transpose(lt)[0:1, :] + bo_ref[...]
    lg_ref[...] = logits

    col = i * tv + lax.broadcasted_iota(jnp.int32, (1, tv), 1)
    lm = jnp.where(col < vocab, logits, -jnp.inf)
    m_new = jnp.maximum(m_ref[0], jnp.max(lm))
    scale = jnp.max(jnp.exp(jnp.full((1, 128), m_ref[0] - m_new)))
    s_ref[0] = s_ref[0] * scale + jnp.sum(jnp.exp(lm - m_new))
    m_ref[0] = m_new

    @pl.when(i == nt - 1)
    def _finish():
        lse_ref[...] = jnp.full((1, 1), m_ref[0]) + jnp.log(
            jnp.full((1, 1), s_ref[0]))


def _tc_logits_lse(pT, wpT, bpT, woT, bo2):
    vocab, d = woT.shape
    tv = 8192
    nt = -(-vocab // tv)

    return pl.pallas_call(
        functools.partial(_a_body, nt, tv, vocab),
        grid=(nt,),
        in_specs=[
            pl.BlockSpec((d, _NW), lambda i: (0, 0)),
            pl.BlockSpec((d, d), lambda i: (0, 0)),
            pl.BlockSpec((d, 1), lambda i: (0, 0)),
            pl.BlockSpec((tv, d), lambda i: (i, 0)),
            pl.BlockSpec((1, tv), lambda i: (0, i)),
        ],
        out_specs=[
            pl.BlockSpec((1, tv), lambda i: (0, i)),
            pl.BlockSpec((1, 1), lambda i: (0, 0)),
        ],
        out_shape=[
            jax.ShapeDtypeStruct((1, vocab), jnp.float32),
            jax.ShapeDtypeStruct((1, 1), jnp.float32),
        ],
        scratch_shapes=[
            pltpu.VMEM((d, 8), jnp.float32),
            pltpu.SMEM((1,), jnp.float32),
            pltpu.SMEM((1,), jnp.float32),
        ],
        compiler_params=pltpu.CompilerParams(
            dimension_semantics=("arbitrary",),
        ),
    )(pT, wpT, bpT, woT, bo2)


def _b_body(lg_ref, lse_ref, out_ref):
    out_ref[...] = lg_ref[...] - lse_ref[...]


def _tc_subtract(logits, lse):
    vocab = logits.shape[1]
    tv = 8192
    nt = -(-vocab // tv)
    return pl.pallas_call(
        _b_body,
        grid=(nt,),
        in_specs=[
            pl.BlockSpec((1, tv), lambda i: (0, i)),
            pl.BlockSpec((1, 1), lambda i: (0, 0)),
        ],
        out_specs=pl.BlockSpec((1, tv), lambda i: (0, i)),
        out_shape=jax.ShapeDtypeStruct((1, vocab), jnp.float32),
    )(logits, lse)


def kernel(inputs, table, W_proj, b_proj, W_out, b_out):
    n = inputs.shape[0]
    idx = inputs.astype(jnp.int32)
    n_pad = _NW * _GROUP
    idx_pad = jnp.pad(idx, (0, n_pad - n))
    partials = _sc_gather_sum(idx_pad, table, n)
    logits, lse = _tc_logits_lse(
        partials.T,
        W_proj.T,
        b_proj.reshape(-1, 1),
        W_out.T,
        b_out.reshape(1, -1),
    )
    return _tc_subtract(logits, lse)


# speedup vs baseline: 1.8750x; 1.1605x over previous
import functools

import jax
import jax.numpy as jnp
from jax import lax
from jax.experimental import pallas as pl
from jax.experimental.pallas import tpu as pltpu
from jax.experimental.pallas import tpu_sc as plsc

_NC = 2
_NS = 16
_NW = _NC * _NS
_GROUP = 8
_LANES = 16


def _sc_gather_sum(idx_pad, table, n_valid):
    vocab, d = table.shape
    mesh = plsc.VectorSubcoreMesh(core_axis_name="c", subcore_axis_name="s")

    def body(idx_hbm, table_hbm, out_hbm, idx_v, rows_v, acc_v, sem):
        wid = lax.axis_index("s") * _NC + lax.axis_index("c")
        base = wid * _GROUP
        pltpu.sync_copy(idx_hbm.at[pl.ds(base, _GROUP)], idx_v)
        pltpu.async_copy(table_hbm.at[idx_v], rows_v, sem).wait()
        for c in range(d // _LANES):
            sl = pl.ds(c * _LANES, _LANES)
            acc = jnp.zeros((_LANES,), jnp.float32)
            for r in range(_GROUP):
                w_r = jnp.where(base + r < n_valid, 1.0, 0.0)
                acc = acc + rows_v[r, sl] * w_r
            acc_v[sl] = acc
        pltpu.sync_copy(acc_v, out_hbm.at[wid])

    run = pl.kernel(
        body,
        out_type=jax.ShapeDtypeStruct((_NW, d), jnp.float32),
        mesh=mesh,
        scratch_types=[
            pltpu.VMEM((_GROUP,), jnp.int32),
            pltpu.VMEM((_GROUP, d), jnp.float32),
            pltpu.VMEM((d,), jnp.float32),
            pltpu.SemaphoreType.DMA,
        ],
    )
    return run(idx_pad, table)


def _a_body(nt, tv, vocab, pT_ref, wpT_ref, bpT_ref, wo_ref, bo_ref,
            lg_ref, lse_ref, h_sc, m_ref, s_ref):
    i = pl.program_id(0)

    @pl.when(i == 0)
    def _init():
        e = jnp.sum(pT_ref[...], axis=1, keepdims=True)
        e8 = jnp.broadcast_to(e, (e.shape[0], 8))
        h = jnp.dot(wpT_ref[...], e8, preferred_element_type=jnp.float32)
        h_sc[...] = jnp.maximum(h + bpT_ref[...], 0.0)
        m_ref[0] = -jnp.inf
        s_ref[0] = 0.0

    lt = jnp.dot(wo_ref[...], h_sc[...], preferred_element_type=jnp.float32)
    logits = jnp.transpose(lt)[0:1, :] + bo_ref[...]
    lg_ref[...] = logits

    col = i * tv + lax.broadcasted_iota(jnp.int32, (1, tv), 1)
    lm = jnp.where(col < vocab, logits, -jnp.inf)
    m_new = jnp.maximum(m_ref[0], jnp.max(lm))
    scale = jnp.max(jnp.exp(jnp.full((1, 128), m_ref[0] - m_new)))
    s_ref[0] = s_ref[0] * scale + jnp.sum(jnp.exp(lm - m_new))
    m_ref[0] = m_new

    @pl.when(i == nt - 1)
    def _finish():
        lse_ref[...] = jnp.full((1, 1), m_ref[0]) + jnp.log(
            jnp.full((1, 1), s_ref[0]))


def _tc_logits_lse(pT, wpT, bpT, woT, bo2):
    vocab, d = woT.shape
    tv = 16384
    nt = -(-vocab // tv)

    return pl.pallas_call(
        functools.partial(_a_body, nt, tv, vocab),
        grid=(nt,),
        in_specs=[
            pl.BlockSpec((d, _NW), lambda i: (0, 0)),
            pl.BlockSpec((d, d), lambda i: (0, 0)),
            pl.BlockSpec((d, 1), lambda i: (0, 0)),
            pl.BlockSpec((tv, d), lambda i: (i, 0)),
            pl.BlockSpec((1, tv), lambda i: (0, i)),
        ],
        out_specs=[
            pl.BlockSpec((1, tv), lambda i: (0, i)),
            pl.BlockSpec((1, 1), lambda i: (0, 0)),
        ],
        out_shape=[
            jax.ShapeDtypeStruct((1, vocab), jnp.float32),
            jax.ShapeDtypeStruct((1, 1), jnp.float32),
        ],
        scratch_shapes=[
            pltpu.VMEM((d, 8), jnp.float32),
            pltpu.SMEM((1,), jnp.float32),
            pltpu.SMEM((1,), jnp.float32),
        ],
        compiler_params=pltpu.CompilerParams(
            dimension_semantics=("arbitrary",),
        ),
    )(pT, wpT, bpT, woT, bo2)


def _b_body(lg_ref, lse_ref, out_ref):
    out_ref[...] = lg_ref[...] - lse_ref[...]


def _tc_subtract(logits, lse):
    vocab = logits.shape[1]
    return pl.pallas_call(
        _b_body,
        in_specs=[
            pl.BlockSpec((1, vocab), lambda: (0, 0)),
            pl.BlockSpec((1, 1), lambda: (0, 0)),
        ],
        out_specs=pl.BlockSpec((1, vocab), lambda: (0, 0)),
        out_shape=jax.ShapeDtypeStruct((1, vocab), jnp.float32),
    )(logits, lse)


def kernel(inputs, table, W_proj, b_proj, W_out, b_out):
    n = inputs.shape[0]
    idx = inputs.astype(jnp.int32)
    n_pad = _NW * _GROUP
    idx_pad = jnp.pad(idx, (0, n_pad - n))
    partials = _sc_gather_sum(idx_pad, table, n)
    logits, lse = _tc_logits_lse(
        partials.T,
        W_proj.T,
        b_proj.reshape(-1, 1),
        W_out.T,
        b_out.reshape(1, -1),
    )
    return _tc_subtract(logits, lse)
